# baseline (speedup 1.0000x reference)
import jax
import jax.numpy as jnp
from jax.experimental import pallas as pl


def _mlp_block(x_ref, w1_ref, b1_ref, w2_ref, b2_ref, o_ref):
    x = x_ref[...]
    h = jnp.maximum(x * w1_ref[...] + b1_ref[...], 0.0)
    o_ref[...] = (
        jnp.dot(h, w2_ref[...], preferred_element_type=jnp.float32)
        + b2_ref[...]
    )


def kernel(rx, phase, W1, b1, W2, b2):
    del phase
    N = rx.shape[0]
    H = W1.shape[1]
    S = W2.shape[1]
    Hp = 128
    W1p = jnp.zeros((1, Hp), jnp.float32).at[:, :H].set(W1)
    b1p = jnp.zeros((1, Hp), jnp.float32).at[:, :H].set(b1)
    W2p = jnp.zeros((Hp, S), jnp.float32).at[:H, :].set(W2)
    b2p = b2.reshape(1, S)

    B = 4096
    out = pl.pallas_call(
        _mlp_block,
        grid=(N // B,),
        in_specs=[
            pl.BlockSpec((B, 1), lambda i: (i, 0)),
            pl.BlockSpec((1, Hp), lambda i: (0, 0)),
            pl.BlockSpec((1, Hp), lambda i: (0, 0)),
            pl.BlockSpec((Hp, S), lambda i: (0, 0)),
            pl.BlockSpec((1, S), lambda i: (0, 0)),
        ],
        out_specs=pl.BlockSpec((B, S), lambda i: (i, 0)),
        out_shape=jax.ShapeDtypeStruct((N, S), jnp.float32),
    )(rx, W1p, b1p, W2p, b2p)
    return out
